# baseline (speedup 1.0000x reference)
import functools

import jax
import jax.numpy as jnp
from jax import lax
from jax.experimental import pallas as pl
from jax.experimental.pallas import tpu as pltpu
from jax.experimental.pallas import tpu_sc as plsc

N_WORKERS = 32


@functools.lru_cache(maxsize=None)
def _build(B, V, D):
    b_per_w = B // N_WORKERS
    slab = D * b_per_w
    lane = 128
    n_c = b_per_w // lane
    mesh = plsc.VectorSubcoreMesh(core_axis_name="c", subcore_axis_name="s")

    @functools.partial(
        pl.kernel,
        mesh=mesh,
        out_type=jax.ShapeDtypeStruct((B * D,), jnp.float32),
        scratch_types=[
            pltpu.VMEM((b_per_w,), jnp.int32),
            pltpu.VMEM((slab,), jnp.float32),
            pltpu.SemaphoreType.DMA,
        ],
        compiler_params=pltpu.CompilerParams(use_tc_tiling_on_sc=False),
    )
    def k(idx_hbm, wt_hbm, out_hbm, idx_v, data_v, sem):
        wid = lax.axis_index("s") * 2 + lax.axis_index("c")
        base = wid * b_per_w
        pltpu.sync_copy(idx_hbm.at[pl.ds(base, b_per_w)], idx_v)

        for d in range(D):
            row = wt_hbm.at[d]
            for c in range(n_c):
                pltpu.async_copy(
                    row.at[idx_v.at[pl.ds(c * lane, lane)]],
                    data_v.at[pl.ds(d * b_per_w + c * lane, lane)],
                    sem,
                )
        pltpu.make_async_copy(
            wt_hbm.at[0].at[pl.ds(0, slab)], data_v, sem
        ).wait()

        pltpu.sync_copy(data_v, out_hbm.at[pl.ds(wid * slab, slab)])

    return k


def kernel(input, weight):
    B = input.shape[0]
    V, D = weight.shape
    idx = input.astype(jnp.int32)
    flat = _build(B, V, D)(idx, weight.T)
    arr = flat.reshape(N_WORKERS, D, B // N_WORKERS)
    return jnp.transpose(arr, (0, 2, 1)).reshape(B, D)


# speedup vs baseline: 4.7110x; 4.7110x over previous
import functools

import jax
import jax.numpy as jnp
from jax import lax
from jax.experimental import pallas as pl
from jax.experimental.pallas import tpu as pltpu
from jax.experimental.pallas import tpu_sc as plsc

N_WORKERS = 32
LANES = 16
PACK = 4


@functools.lru_cache(maxsize=None)
def _build(B, V, D):
    b_per_w = B // N_WORKERS
    n_q = b_per_w // LANES
    lane = 128
    n_c = b_per_w // lane
    mesh = plsc.VectorSubcoreMesh(core_axis_name="c", subcore_axis_name="s")

    @functools.partial(
        pl.kernel,
        mesh=mesh,
        out_type=jax.ShapeDtypeStruct((B, D), jnp.float32),
        scratch_types=[
            pltpu.VMEM((b_per_w,), jnp.int32),
            pltpu.VMEM((b_per_w,), jnp.int32),
            pltpu.VMEM((128, PACK * D), jnp.float32),
            pltpu.VMEM((b_per_w, D), jnp.float32),
            pltpu.SemaphoreType.DMA,
        ],
        compiler_params=pltpu.CompilerParams(
            use_tc_tiling_on_sc=True, needs_layout_passes=False
        ),
    )
    def k(idx_hbm, tab_hbm, out_hbm, rid_v, qoff_v, rows_v, out_v, sem):
        wid = lax.axis_index("s") * 2 + lax.axis_index("c")
        base = wid * b_per_w
        pltpu.sync_copy(idx_hbm.at[pl.ds(base, b_per_w)], rid_v)

        for q in range(n_q):
            v = rid_v[pl.ds(q * LANES, LANES)]
            rid_v[pl.ds(q * LANES, LANES)] = v >> 2
            qoff_v[pl.ds(q * LANES, LANES)] = (v & 3) * D

        for c in range(n_c):
            pltpu.async_copy(
                tab_hbm.at[rid_v.at[pl.ds(c * lane, lane)]],
                rows_v,
                sem,
            ).wait()
            for ql in range(lane // LANES):
                q = c * (lane // LANES) + ql
                jv = lax.iota(jnp.int32, LANES) + ql * LANES
                gjv = jv + c * lane
                mv = qoff_v[pl.ds(q * LANES, LANES)]
                for d in range(D):
                    vals = plsc.load_gather(rows_v, [jv, mv + d])
                    plsc.store_scatter(
                        out_v, [gjv, jnp.full((LANES,), d, jnp.int32)], vals
                    )

        pltpu.sync_copy(out_v, out_hbm.at[pl.ds(base, b_per_w), :])

    return k


def kernel(input, weight):
    B = input.shape[0]
    V, D = weight.shape
    idx = input.astype(jnp.int32)
    tab = weight.reshape(V // PACK, PACK * D)
    return _build(B, V, D)(idx, tab)


# speedup vs baseline: 4.9303x; 1.0466x over previous
import functools

import jax
import jax.numpy as jnp
from jax import lax
from jax.experimental import pallas as pl
from jax.experimental.pallas import tpu as pltpu
from jax.experimental.pallas import tpu_sc as plsc

N_WORKERS = 32
CHUNK = 128


@functools.lru_cache(maxsize=None)
def _build(B, V, D):
    b_per_w = B // N_WORKERS
    n_chunks = b_per_w // CHUNK
    mesh = plsc.VectorSubcoreMesh(core_axis_name="c", subcore_axis_name="s")

    @functools.partial(
        pl.kernel,
        mesh=mesh,
        out_type=jax.ShapeDtypeStruct((B, D), jnp.float32),
        scratch_types=[
            pltpu.VMEM((n_chunks, CHUNK), jnp.int32),
            pltpu.VMEM((b_per_w, D), jnp.float32),
            pltpu.SemaphoreType.DMA,
        ],
        compiler_params=pltpu.CompilerParams(use_tc_tiling_on_sc=False),
    )
    def k(idx_hbm, table_hbm, out_hbm, idx_v, rows_v, sem):
        wid = lax.axis_index("s") * 2 + lax.axis_index("c")
        base = wid * n_chunks
        pltpu.sync_copy(idx_hbm.at[pl.ds(base, n_chunks)], idx_v)
        copies = []
        for j in range(n_chunks):
            copies.append(
                pltpu.async_copy(
                    table_hbm.at[idx_v.at[j]],
                    rows_v.at[pl.ds(j * CHUNK, CHUNK)],
                    sem,
                )
            )
        for c in copies:
            c.wait()
        pltpu.sync_copy(rows_v, out_hbm.at[pl.ds(base * CHUNK, b_per_w)])

    return k


def kernel(input, weight):
    B = input.shape[0]
    V, D = weight.shape
    idx = input.astype(jnp.int32).reshape(B // CHUNK, CHUNK)
    return _build(B, V, D)(idx, weight)
